# baseline (speedup 1.0000x reference)
import functools

import jax
import jax.numpy as jnp
from jax import lax
from jax.experimental import pallas as pl
from jax.experimental.pallas import tpu as pltpu
from jax.experimental.pallas import tpu_sc as plsc

H = 512
W = 512
NS = 16
L = 16
RROWS = H // NS
NCHUNK = 4
CROWS = RROWS // NCHUNK
WVECS = W // L
PAIRS = RROWS // 2

_mesh = plsc.VectorSubcoreMesh(
    core_axis_name="c", subcore_axis_name="s", num_cores=1)


@functools.partial(
    pl.kernel,
    mesh=_mesh,
    out_type=jax.ShapeDtypeStruct((H, W), jnp.float32),
    scratch_types=[
        [pltpu.VMEM((CROWS, W), jnp.int32) for _ in range(NCHUNK)],
        pltpu.VMEM((2, W), jnp.float32),
        pltpu.VMEM((2, W), jnp.float32),
        pltpu.VMEM((NS * L,), jnp.int32),
        pltpu.VMEM_SHARED((NS * L,), jnp.int32),
        [pltpu.SemaphoreType.DMA for _ in range(NCHUNK)],
        pltpu.SemaphoreType.DMA,
    ],
)
def _sampler_sc(label_hbm, out_hbm, chunks, val_v, ones_v,
                part_v, shared, in_sems, sem_o):
    s = lax.axis_index("s")
    row0 = s * RROWS

    in_cps = [
        pltpu.async_copy(
            label_hbm.at[pl.ds(row0 + k * CROWS, CROWS), :], chunks[k],
            in_sems[k])
        for k in range(NCHUNK)
    ]

    one = jnp.full((L,), 1.0, jnp.float32)
    for r in range(2):
        for j in range(WVECS):
            ones_v[r, pl.ds(j * L, L)] = one

    m = jnp.zeros((L,), jnp.int32)
    for k in range(NCHUNK):
        in_cps[k].wait()

        @plsc.parallel_loop(0, WVECS, carry=m)
        def m(j, mm, _ref=chunks[k]):
            v = [_ref[r, pl.ds(j * L, L)] for r in range(CROWS)]
            stride = CROWS // 2
            while stride >= 1:
                v = [jnp.maximum(v[t], v[t + stride]) for t in range(stride)]
                stride //= 2
            return jnp.maximum(mm, v[0])

    part_v[pl.ds(0, L)] = m
    pltpu.sync_copy(part_v.at[pl.ds(0, L)], shared.at[pl.ds(s * L, L)])
    plsc.subcore_barrier()
    pltpu.sync_copy(shared, part_v)

    v2 = [part_v[pl.ds(j * L, L)] for j in range(NS)]
    stride = NS // 2
    while stride >= 1:
        v2 = [jnp.maximum(v2[k], v2[k + stride]) for k in range(stride)]
        stride //= 2
    acc2 = v2[0]

    gmax = acc2[0]
    for i in range(1, L):
        gmax = jnp.maximum(gmax, acc2[i])

    val = jnp.where(gmax >= 11, 0.0, 1.0).astype(jnp.float32)
    vec = jnp.full((L,), val, jnp.float32)

    for r in range(2):
        for j in range(WVECS):
            val_v[r, pl.ds(j * L, L)] = vec

    copies = [
        pltpu.async_copy(val_v, out_hbm.at[pl.ds(row0 + 2 * p, 2), :],
                         sem_o)
        for p in range(PAIRS)
    ]
    for cp in copies:
        cp.wait()

    @pl.when(s == 0)
    def _():
        pltpu.sync_copy(ones_v, out_hbm.at[pl.ds(0, 2), :])


def kernel(label):
    return _sampler_sc(label)


# speedup vs baseline: 1.0197x; 1.0197x over previous
import functools

import jax
import jax.numpy as jnp
from jax import lax
from jax.experimental import pallas as pl
from jax.experimental.pallas import tpu as pltpu
from jax.experimental.pallas import tpu_sc as plsc

H = 512
W = 512
NS = 16
L = 16
RROWS = H // NS
HALF = RROWS // 2
WVECS = W // L
PAIRS = RROWS // 2

_mesh = plsc.VectorSubcoreMesh(
    core_axis_name="c", subcore_axis_name="s", num_cores=1)


def _band_max(ref, m):
    @plsc.parallel_loop(0, WVECS, carry=m)
    def acc(j, mm):
        v = [ref[r, pl.ds(j * L, L)] for r in range(HALF)]
        stride = HALF // 2
        while stride >= 1:
            v = [jnp.maximum(v[k], v[k + stride]) for k in range(stride)]
            stride //= 2
        return jnp.maximum(mm, v[0])

    return acc


@functools.partial(
    pl.kernel,
    mesh=_mesh,
    out_type=jax.ShapeDtypeStruct((H, W), jnp.float32),
    scratch_types=[
        pltpu.VMEM((HALF, W), jnp.int32),
        pltpu.VMEM((HALF, W), jnp.int32),
        pltpu.VMEM((2, W), jnp.float32),
        pltpu.VMEM((2, W), jnp.float32),
        pltpu.VMEM((NS * L,), jnp.int32),
        pltpu.VMEM_SHARED((NS * L,), jnp.int32),
        pltpu.SemaphoreType.DMA,
        pltpu.SemaphoreType.DMA,
        pltpu.SemaphoreType.DMA,
    ],
)
def _sampler_sc(label_hbm, out_hbm, chunk_a, chunk_b, val_v, ones_v,
                part_v, shared, sem_a, sem_b, sem_o):
    s = lax.axis_index("s")
    row0 = s * RROWS

    cp_a = pltpu.async_copy(label_hbm.at[pl.ds(row0, HALF), :], chunk_a,
                            sem_a)
    cp_b = pltpu.async_copy(label_hbm.at[pl.ds(row0 + HALF, HALF), :],
                            chunk_b, sem_b)
    cp_a.wait()
    m = _band_max(chunk_a, jnp.zeros((L,), jnp.int32))
    cp_b.wait()
    m = _band_max(chunk_b, m)

    part_v[pl.ds(0, L)] = m
    pltpu.sync_copy(part_v.at[pl.ds(0, L)], shared.at[pl.ds(s * L, L)])
    plsc.subcore_barrier()
    pltpu.sync_copy(shared, part_v)

    v2 = [part_v[pl.ds(j * L, L)] for j in range(NS)]
    stride = NS // 2
    while stride >= 1:
        v2 = [jnp.maximum(v2[k], v2[k + stride]) for k in range(stride)]
        stride //= 2
    acc2 = v2[0]

    gmax = acc2[0]
    for i in range(1, L):
        gmax = jnp.maximum(gmax, acc2[i])

    val = jnp.where(gmax >= 11, 0.0, 1.0).astype(jnp.float32)
    vec = jnp.full((L,), val, jnp.float32)
    ones = jnp.full((L,), 1.0, jnp.float32)

    for r in range(2):
        for j in range(WVECS):
            val_v[r, pl.ds(j * L, L)] = vec
            ones_v[r, pl.ds(j * L, L)] = ones

    copies = [
        pltpu.async_copy(val_v, out_hbm.at[pl.ds(row0 + 2 * p, 2), :],
                         sem_o)
        for p in range(PAIRS)
    ]
    for cp in copies:
        cp.wait()

    @pl.when(s == 0)
    def _():
        pltpu.sync_copy(ones_v, out_hbm.at[pl.ds(0, 2), :])


def kernel(label):
    return _sampler_sc(label)
